# baseline (speedup 1.0000x reference)
import functools

import jax
import jax.numpy as jnp
from jax import lax
from jax.experimental import pallas as pl
from jax.experimental.pallas import tpu as pltpu
from jax.experimental.pallas import tpu_sc as plsc

B, H, N, F_IN, F_OUT = 8, 12, 1024, 192, 192
DH = F_OUT // H
BN = 256

_NC, _NS = 2, 16
_NW = _NC * _NS
_SLABS_PER_W = (B * H) // _NW


def _gcn_body(x_ref, adj_ref, w_ref, b_ref, h_ref, support_ref):
    i = pl.program_id(1)

    @pl.when(i == 0)
    def _():
        support_ref[...] = jnp.dot(
            x_ref[0].astype(jnp.bfloat16),
            w_ref[...].astype(jnp.bfloat16),
            preferred_element_type=jnp.float32,
        )

    parts = []
    for h in range(H):
        a = adj_ref[0, h].astype(jnp.bfloat16)
        s = support_ref[:, h * DH:(h + 1) * DH]
        parts.append(
            jnp.dot(a, s.astype(jnp.bfloat16),
                    preferred_element_type=jnp.float32)
        )
    acc = jnp.concatenate(parts, axis=-1)
    h_ref[0] = jax.nn.gelu(acc + b_ref[...])


def _tc_compute(x, adj, W, b2):
    grid = (B, N // BN)
    return pl.pallas_call(
        _gcn_body,
        grid=grid,
        in_specs=[
            pl.BlockSpec((1, N, F_IN), lambda bi, i: (bi, 0, 0)),
            pl.BlockSpec((1, H, BN, N), lambda bi, i: (bi, 0, i, 0)),
            pl.BlockSpec((F_IN, F_OUT), lambda bi, i: (0, 0)),
            pl.BlockSpec((1, F_OUT), lambda bi, i: (0, 0)),
        ],
        out_specs=pl.BlockSpec((1, BN, F_OUT), lambda bi, i: (bi, i, 0)),
        out_shape=jax.ShapeDtypeStruct((B, N, F_OUT), jnp.float32),
        scratch_shapes=[pltpu.VMEM((N, F_OUT), jnp.float32)],
    )(x, adj, W, b2)


def _sc_copy(adj3):
    mesh = plsc.VectorSubcoreMesh(core_axis_name="c", subcore_axis_name="s")

    @functools.partial(
        pl.kernel,
        mesh=mesh,
        out_type=jax.ShapeDtypeStruct((B * H, N, N), jnp.float32),
        scratch_types=[pltpu.SemaphoreType.DMA],
    )
    def copy_kernel(adj_hbm, out_hbm, sem):
        wid = lax.axis_index("s") * _NC + lax.axis_index("c")
        base = wid * _SLABS_PER_W
        copies = [
            pltpu.make_async_copy(adj_hbm.at[base + j], out_hbm.at[base + j], sem)
            for j in range(_SLABS_PER_W)
        ]
        for c in copies:
            c.start()
        for c in copies:
            c.wait()

    return copy_kernel(adj3)


@jax.jit
def kernel(x, adj, W, b):
    b2 = b.reshape(1, F_OUT)
    h_out = _tc_compute(x, adj, W, b2)
    adjc = _sc_copy(adj.reshape(B * H, N, N))
    return h_out, adjc


# speedup vs baseline: 28.2309x; 28.2309x over previous
import functools

import jax
import jax.numpy as jnp
from jax import lax
from jax.experimental import pallas as pl
from jax.experimental.pallas import tpu as pltpu
from jax.experimental.pallas import tpu_sc as plsc

B, H, N, F_IN, F_OUT = 8, 12, 1024, 192, 192
DH = F_OUT // H
BN = 256

_NC, _NS = 2, 16
_NW = _NC * _NS
_ROWS = B * H * N
_RPW = _ROWS // _NW
_CH = 48
_NCH = _RPW // _CH


def _gcn_body(x_ref, adj_ref, w_ref, b_ref, h_ref, support_ref):
    i = pl.program_id(1)

    @pl.when(i == 0)
    def _():
        support_ref[...] = jnp.dot(
            x_ref[0].astype(jnp.bfloat16),
            w_ref[...].astype(jnp.bfloat16),
            preferred_element_type=jnp.float32,
        )

    parts = []
    for h in range(H):
        a = adj_ref[0, h].astype(jnp.bfloat16)
        s = support_ref[:, h * DH:(h + 1) * DH]
        parts.append(
            jnp.dot(a, s.astype(jnp.bfloat16),
                    preferred_element_type=jnp.float32)
        )
    acc = jnp.concatenate(parts, axis=-1)
    h_ref[0] = jax.nn.gelu(acc + b_ref[...])


def _tc_compute(x, adj, W, b2):
    grid = (B, N // BN)
    return pl.pallas_call(
        _gcn_body,
        grid=grid,
        in_specs=[
            pl.BlockSpec((1, N, F_IN), lambda bi, i: (bi, 0, 0)),
            pl.BlockSpec((1, H, BN, N), lambda bi, i: (bi, 0, i, 0)),
            pl.BlockSpec((F_IN, F_OUT), lambda bi, i: (0, 0)),
            pl.BlockSpec((1, F_OUT), lambda bi, i: (0, 0)),
        ],
        out_specs=pl.BlockSpec((1, BN, F_OUT), lambda bi, i: (bi, i, 0)),
        out_shape=jax.ShapeDtypeStruct((B, N, F_OUT), jnp.float32),
        scratch_shapes=[pltpu.VMEM((N, F_OUT), jnp.float32)],
    )(x, adj, W, b2)


def _sc_copy(adj2):
    mesh = plsc.VectorSubcoreMesh(core_axis_name="c", subcore_axis_name="s")

    @functools.partial(
        pl.kernel,
        mesh=mesh,
        out_type=jax.ShapeDtypeStruct((_ROWS, N), jnp.float32),
        scratch_types=[
            pltpu.VMEM((_CH, N), jnp.float32),
            pltpu.VMEM((_CH, N), jnp.float32),
            pltpu.SemaphoreType.DMA,
        ],
    )
    def copy_kernel(adj_hbm, out_hbm, b0, b1, sem):
        wid = lax.axis_index("s") * _NC + lax.axis_index("c")
        base = wid * _RPW
        bufs = (b0, b1)

        def rd(k, buf):
            return pltpu.make_async_copy(
                adj_hbm.at[pl.ds(base + k * _CH, _CH)], buf, sem)

        rd(0, b0).start()
        rd(1, b1).start()

        def body(g, carry):
            for t in range(2):
                k = 2 * g + t
                buf = bufs[t]
                rd(k, buf).wait()
                pltpu.sync_copy(buf, out_hbm.at[pl.ds(base + k * _CH, _CH)])

                @pl.when(k + 2 < _NCH)
                def _():
                    rd(k + 2, buf).start()

            return carry

        lax.fori_loop(0, _NCH // 2, body, 0)

    return copy_kernel(adj2)


@jax.jit
def kernel(x, adj, W, b):
    b2 = b.reshape(1, F_OUT)
    h_out = _tc_compute(x, adj, W, b2)
    adjc = _sc_copy(adj.reshape(_ROWS, N))
    return h_out, adjc.reshape(B * H, N, N)


# speedup vs baseline: 28.3548x; 1.0044x over previous
import functools

import jax
import jax.numpy as jnp
from jax import lax
from jax.experimental import pallas as pl
from jax.experimental.pallas import tpu as pltpu
from jax.experimental.pallas import tpu_sc as plsc

B, H, N, F_IN, F_OUT = 8, 12, 1024, 192, 192
DH = F_OUT // H
BN = 256

_NC, _NS = 2, 16
_NW = _NC * _NS
_ROWS = B * H * N
_RPW = _ROWS // _NW
_CH = 32
_NCH = _RPW // _CH


def _gcn_body(x_ref, adj_ref, w_ref, b_ref, h_ref, support_ref):
    i = pl.program_id(1)

    @pl.when(i == 0)
    def _():
        support_ref[...] = jnp.dot(
            x_ref[0].astype(jnp.bfloat16),
            w_ref[...].astype(jnp.bfloat16),
            preferred_element_type=jnp.float32,
        )

    parts = []
    for h in range(H):
        a = adj_ref[0, h].astype(jnp.bfloat16)
        s = support_ref[:, h * DH:(h + 1) * DH]
        parts.append(
            jnp.dot(a, s.astype(jnp.bfloat16),
                    preferred_element_type=jnp.float32)
        )
    acc = jnp.concatenate(parts, axis=-1)
    h_ref[0] = jax.nn.gelu(acc + b_ref[...])


def _tc_compute(x, adj, W, b2):
    grid = (B, N // BN)
    return pl.pallas_call(
        _gcn_body,
        grid=grid,
        in_specs=[
            pl.BlockSpec((1, N, F_IN), lambda bi, i: (bi, 0, 0)),
            pl.BlockSpec((1, H, BN, N), lambda bi, i: (bi, 0, i, 0)),
            pl.BlockSpec((F_IN, F_OUT), lambda bi, i: (0, 0)),
            pl.BlockSpec((1, F_OUT), lambda bi, i: (0, 0)),
        ],
        out_specs=pl.BlockSpec((1, BN, F_OUT), lambda bi, i: (bi, i, 0)),
        out_shape=jax.ShapeDtypeStruct((B, N, F_OUT), jnp.float32),
        scratch_shapes=[pltpu.VMEM((N, F_OUT), jnp.float32)],
    )(x, adj, W, b2)


def _sc_copy(adj2):
    mesh = plsc.VectorSubcoreMesh(core_axis_name="c", subcore_axis_name="s")

    @functools.partial(
        pl.kernel,
        mesh=mesh,
        out_type=jax.ShapeDtypeStruct((_ROWS, N), jnp.float32),
        scratch_types=[
            pltpu.VMEM((_CH, N), jnp.float32),
            pltpu.VMEM((_CH, N), jnp.float32),
            pltpu.VMEM((_CH, N), jnp.float32),
            pltpu.SemaphoreType.DMA,
            pltpu.SemaphoreType.DMA,
        ],
    )
    def copy_kernel(adj_hbm, out_hbm, b0, b1, b2, sem_r, sem_w):
        wid = lax.axis_index("s") * _NC + lax.axis_index("c")
        base = wid * _RPW
        bufs = (b0, b1, b2)

        def rd(k, m):
            return pltpu.make_async_copy(
                adj_hbm.at[pl.ds(base + k * _CH, _CH)], bufs[m], sem_r)

        def wr(k, m):
            return pltpu.make_async_copy(
                bufs[m], out_hbm.at[pl.ds(base + k * _CH, _CH)], sem_w)

        rd(0, 0).start()
        rd(1, 1).start()

        def body(g, carry):
            for t in range(3):
                k = 3 * g + t
                rd(k, t).wait()
                wr(k, t).start()

                @pl.when(k >= 1)
                def _():
                    wr(k - 1, (t - 1) % 3).wait()

                @pl.when(k + 2 < _NCH)
                def _():
                    rd(k + 2, (t + 2) % 3).start()

            return carry

        lax.fori_loop(0, _NCH // 3, body, 0)
        wr(_NCH - 1, (_NCH - 1) % 3).wait()

    return copy_kernel(adj2)


@jax.jit
def kernel(x, adj, W, b):
    b2 = b.reshape(1, F_OUT)
    h_out = _tc_compute(x, adj, W, b2)
    adjc = _sc_copy(adj.reshape(_ROWS, N))
    return h_out, adjc.reshape(B * H, N, N)


# speedup vs baseline: 44.4425x; 1.5674x over previous
import jax
import jax.numpy as jnp
from jax.experimental import pallas as pl
from jax.experimental.pallas import tpu as pltpu

B, H, N, F_IN, F_OUT = 8, 12, 1024, 192, 192
DH = F_OUT // H
BN = 256


def _gcn_body(x_ref, adj_ref, w_ref, b_ref, h_ref, adjc_ref, support_ref, sem):
    i = pl.program_id(1)

    @pl.when(i == 0)
    def _():
        support_ref[...] = jnp.dot(
            x_ref[0].astype(jnp.bfloat16),
            w_ref[...].astype(jnp.bfloat16),
            preferred_element_type=jnp.float32,
        )

    cp = pltpu.make_async_copy(adj_ref, adjc_ref, sem)
    cp.start()

    parts = []
    for h in range(H):
        a = adj_ref[0, h].astype(jnp.bfloat16)
        s = support_ref[:, h * DH:(h + 1) * DH]
        parts.append(
            jnp.dot(a, s.astype(jnp.bfloat16),
                    preferred_element_type=jnp.float32)
        )
    acc = jnp.concatenate(parts, axis=-1)
    h_ref[0] = jax.nn.gelu(acc + b_ref[...])
    cp.wait()


@jax.jit
def kernel(x, adj, W, b):
    b2 = b.reshape(1, F_OUT)
    grid = (B, N // BN)
    h_out, adjc = pl.pallas_call(
        _gcn_body,
        grid=grid,
        in_specs=[
            pl.BlockSpec((1, N, F_IN), lambda bi, i: (bi, 0, 0)),
            pl.BlockSpec((1, H, BN, N), lambda bi, i: (bi, 0, i, 0)),
            pl.BlockSpec((F_IN, F_OUT), lambda bi, i: (0, 0)),
            pl.BlockSpec((1, F_OUT), lambda bi, i: (0, 0)),
        ],
        out_specs=[
            pl.BlockSpec((1, BN, F_OUT), lambda bi, i: (bi, i, 0)),
            pl.BlockSpec((1, H, BN, N), lambda bi, i: (bi, 0, i, 0)),
        ],
        out_shape=[
            jax.ShapeDtypeStruct((B, N, F_OUT), jnp.float32),
            jax.ShapeDtypeStruct((B, H, N, N), jnp.float32),
        ],
        scratch_shapes=[
            pltpu.VMEM((N, F_OUT), jnp.float32),
            pltpu.SemaphoreType.DMA,
        ],
    )(x, adj, W, b2)
    return h_out, adjc.reshape(B * H, N, N)


# speedup vs baseline: 44.7622x; 1.0072x over previous
import jax
import jax.numpy as jnp
from jax.experimental import pallas as pl
from jax.experimental.pallas import tpu as pltpu

B, H, N, F_IN, F_OUT = 8, 12, 1024, 192, 192
DH = F_OUT // H
BN = 512


def _gcn_body(x_ref, adj_ref, w_ref, b_ref, h_ref, adjc_ref, support_ref, sem):
    bi = pl.program_id(0)
    i = pl.program_id(1)

    @pl.when(i == 0)
    def _():
        support_ref[...] = jnp.dot(
            x_ref[0].astype(jnp.bfloat16),
            w_ref[...].astype(jnp.bfloat16),
            preferred_element_type=jnp.float32,
        )

    cp = pltpu.make_async_copy(
        adj_ref,
        adjc_ref.at[pl.ds(bi, 1), :, pl.ds(i * BN, BN), :],
        sem,
    )
    cp.start()

    parts = []
    for h in range(H):
        a = adj_ref[0, h].astype(jnp.bfloat16)
        s = support_ref[:, h * DH:(h + 1) * DH]
        parts.append(
            jnp.dot(a, s.astype(jnp.bfloat16),
                    preferred_element_type=jnp.float32)
        )
    acc = jnp.concatenate(parts, axis=-1)
    h_ref[0] = jax.nn.gelu(acc + b_ref[...])
    cp.wait()


@jax.jit
def kernel(x, adj, W, b):
    b2 = b.reshape(1, F_OUT)
    grid = (B, N // BN)
    h_out, adjc = pl.pallas_call(
        _gcn_body,
        grid=grid,
        in_specs=[
            pl.BlockSpec((1, N, F_IN), lambda bi, i: (bi, 0, 0)),
            pl.BlockSpec((1, H, BN, N), lambda bi, i: (bi, 0, i, 0)),
            pl.BlockSpec((F_IN, F_OUT), lambda bi, i: (0, 0)),
            pl.BlockSpec((1, F_OUT), lambda bi, i: (0, 0)),
        ],
        out_specs=[
            pl.BlockSpec((1, BN, F_OUT), lambda bi, i: (bi, i, 0)),
            pl.BlockSpec(memory_space=pl.ANY),
        ],
        out_shape=[
            jax.ShapeDtypeStruct((B, N, F_OUT), jnp.float32),
            jax.ShapeDtypeStruct((B, H, N, N), jnp.float32),
        ],
        scratch_shapes=[
            pltpu.VMEM((N, F_OUT), jnp.float32),
            pltpu.SemaphoreType.DMA,
        ],
    )(x, adj, W, b2)
    return h_out, adjc.reshape(B * H, N, N)
